# baseline (speedup 1.0000x reference)
import functools

import jax
import jax.numpy as jnp
from jax import lax
from jax.experimental import pallas as pl
from jax.experimental.pallas import tpu as pltpu
from jax.experimental.pallas import tpu_sc as plsc

B, L, EMB, HID, TAGS = 4096, 200, 64, 256, 5
V = 1000000
NC, NS = 2, 16
NW = NC * NS
EX_PER_W = B // NW
TOK_PER_W = EX_PER_W * L
CHUNK = 128
NCHUNK = TOK_PER_W // CHUNK
NQ = 4
EX_PER_Q = EX_PER_W // NQ
NCHUNK_Q = NCHUNK // NQ
NBUF = 5
ACC_ROWS = NS * EX_PER_Q

NUNIT = V // CHUNK
UNIT_TAIL = V - NUNIT * CHUNK
NITER = NUNIT // NW


def _sc_format(tabT, tail_rows):
    mesh = plsc.VectorSubcoreMesh(core_axis_name="c", subcore_axis_name="s")

    @functools.partial(
        pl.kernel,
        out_type=jax.ShapeDtypeStruct((V // 2, 128), jnp.float32),
        mesh=mesh,
        scratch_types=[
            pltpu.VMEM((EMB, CHUNK), jnp.float32),
            pltpu.VMEM((EMB, CHUNK), jnp.float32),
            pltpu.VMEM((EMB, CHUNK), jnp.float32),
            pltpu.VMEM((EMB, CHUNK), jnp.float32),
            *([pltpu.SemaphoreType.DMA] * 4),
        ],
        compiler_params=pltpu.CompilerParams(needs_layout_passes=False),
    )
    def k(tab_hbm, tail_hbm, out_hbm, inb0, inb1, outb0, outb1,
          is0, is1, os0, os1):
        w = lax.axis_index("s") * NC + lax.axis_index("c")

        dim_idx = [lax.iota(jnp.int32, 16) + 16 * kk for kk in range(4)]

        def transpose(inb, outb, nids, off=0):
            @pl.loop(0, nids // 2)
            def _(orow):
                c0 = (off + 2 * orow).astype(jnp.int32)
                for kk in range(4):
                    vL = plsc.load_gather(
                        inb, [dim_idx[kk], jnp.broadcast_to(c0, (16,))])
                    outb.at[orow, pl.ds(16 * kk, 16)][...] = vL
                    vR = plsc.load_gather(
                        inb, [dim_idx[kk], jnp.broadcast_to(c0 + 1, (16,))])
                    outb.at[orow, pl.ds(EMB + 16 * kk, 16)][...] = vR

        def unit_of(i):
            return w + NW * i

        def in_copy(i, inb, sem):
            off = pl.multiple_of(unit_of(i) * CHUNK, CHUNK)
            return pltpu.async_copy(
                tab_hbm.at[:, pl.ds(off, CHUNK)], inb, sem)

        def out_copy(i, outb, sem):
            off = pl.multiple_of(unit_of(i) * (CHUNK // 2), CHUNK // 2)
            return pltpu.async_copy(
                outb, out_hbm.at[pl.ds(off, CHUNK // 2)], sem)

        in_copy(0, inb0, is0)
        in_copy(1, inb1, is1)
        out_copy(0, outb0, os0)
        out_copy(1, outb1, os1)

        @pl.loop(0, NITER, step=2)
        def _(i0):
            for (i_off, inb, outb, isem, osem) in (
                    (0, inb0, outb0, is0, os0), (1, inb1, outb1, is1, os1)):
                i = i0 + i_off
                pltpu.make_async_copy(
                    tab_hbm.at[:, pl.ds(0, CHUNK)], inb, isem).wait()
                pltpu.make_async_copy(
                    outb, out_hbm.at[pl.ds(0, CHUNK // 2)], osem).wait()
                transpose(inb, outb, CHUNK)

                @pl.when(i + 2 < NITER)
                def _():
                    in_copy(i + 2, inb, isem)

                out_copy(i, outb, osem)

        pltpu.make_async_copy(
            outb0, out_hbm.at[pl.ds(0, CHUNK // 2)], os0).wait()
        pltpu.make_async_copy(
            outb1, out_hbm.at[pl.ds(0, CHUNK // 2)], os1).wait()

        u_last = w + NW * NITER

        @pl.when(u_last < NUNIT)
        def _():
            off_in = pl.multiple_of(u_last * CHUNK, CHUNK)
            off_out = pl.multiple_of(u_last * (CHUNK // 2), CHUNK // 2)
            pltpu.sync_copy(tab_hbm.at[:, pl.ds(off_in, CHUNK)], inb0)
            transpose(inb0, outb0, CHUNK)
            pltpu.sync_copy(outb0, out_hbm.at[pl.ds(off_out, CHUNK // 2)])

        @pl.when(u_last == NUNIT)
        def _():
            pltpu.sync_copy(
                tail_hbm,
                out_hbm.at[pl.ds(NUNIT * (CHUNK // 2), UNIT_TAIL // 2)])

    return k(tabT, tail_rows)


def _sc_pooled_sums(x3, seg1, table2):
    mesh = plsc.VectorSubcoreMesh(core_axis_name="c", subcore_axis_name="s")

    @functools.partial(
        pl.kernel,
        out_type=jax.ShapeDtypeStruct((B, EMB), jnp.float32),
        mesh=mesh,
        scratch_types=[
            pltpu.VMEM((NCHUNK, CHUNK), jnp.int32),
            pltpu.VMEM((NCHUNK_Q, CHUNK), jnp.int32),
            pltpu.VMEM((NBUF, CHUNK), jnp.int32),
            pltpu.VMEM((NBUF, CHUNK), jnp.int32),
            pltpu.VMEM((EX_PER_Q, EMB), jnp.float32),
            *([pltpu.VMEM((CHUNK, 128), jnp.float32)] * NBUF),
            pltpu.VMEM_SHARED((2 * ACC_ROWS, 128), jnp.float32),
            *([pltpu.SemaphoreType.DMA] * (2 * NBUF)),
        ],
    )
    def k(x_hbm, seg_hbm, table_hbm, out_hbm, x_v, seg_v, gidx_v, sidx_v,
          sum_v, *rest):
        bufs = rest[:NBUF]
        acc_sh = rest[NBUF]
        gsem = rest[NBUF + 1:NBUF + 1 + NBUF]
        ssem = rest[NBUF + 1 + NBUF:]

        sid = lax.axis_index("s")
        w = sid * NC + lax.axis_index("c")
        pltpu.sync_copy(x_hbm.at[w], x_v)
        pltpu.sync_copy(seg_hbm, seg_v)

        zeros = jnp.zeros((16,), jnp.float32)
        seg_base = (sid * EX_PER_Q).astype(jnp.int32)

        def stage_indices(cx, cs, b):
            @pl.loop(0, CHUNK, step=16)
            def _(j):
                t = x_v.at[cx, pl.ds(j, 16)][...]
                gidx_v.at[b, pl.ds(j, 16)][...] = jax.lax.shift_right_logical(
                    t, 1)
                par = (t & 1) * ACC_ROWS
                sidx_v.at[b, pl.ds(j, 16)][...] = (
                    seg_v.at[cs, pl.ds(j, 16)][...] + par + seg_base)

        def zero_acc():
            @pl.loop(0, EX_PER_Q)
            def _(r):
                @pl.loop(0, 128, step=16)
                def _(j):
                    bufs[0].at[r, pl.ds(j, 16)][...] = zeros
            sl = bufs[0].at[pl.ds(0, EX_PER_Q)]
            pltpu.sync_copy(sl, acc_sh.at[pl.ds(sid * EX_PER_Q, EX_PER_Q)])
            pltpu.sync_copy(
                sl, acc_sh.at[pl.ds(ACC_ROWS + sid * EX_PER_Q, EX_PER_Q)])

        for q in range(NQ):
            zero_acc()
            base = q * NCHUNK_Q
            for b in range(NBUF):
                stage_indices(base + b, b, b)
                pltpu.async_copy(table_hbm.at[gidx_v.at[b]], bufs[b], gsem[b])

            @pl.loop(0, NCHUNK_Q, step=NBUF)
            def _(c0):
                scatters = []
                for b in range(NBUF):
                    pltpu.make_async_copy(
                        table_hbm.at[gidx_v.at[b]], bufs[b], gsem[b]).wait()
                    scatters.append(pltpu.async_copy(
                        bufs[b], acc_sh.at[sidx_v.at[b]], ssem[b], add=True))
                for b in range(NBUF):
                    scatters[b].wait()

                    @pl.when(c0 + NBUF + b < NCHUNK_Q)
                    def _():
                        stage_indices(base + c0 + NBUF + b, c0 + NBUF + b, b)
                        pltpu.async_copy(
                            table_hbm.at[gidx_v.at[b]], bufs[b], gsem[b])

            pltpu.sync_copy(acc_sh.at[pl.ds(sid * EX_PER_Q, EX_PER_Q)],
                            bufs[0].at[pl.ds(0, EX_PER_Q)])
            pltpu.sync_copy(
                acc_sh.at[pl.ds(ACC_ROWS + sid * EX_PER_Q, EX_PER_Q)],
                bufs[1].at[pl.ds(0, EX_PER_Q)])

            @pl.loop(0, EX_PER_Q)
            def _(r):
                @pl.loop(0, EMB, step=16)
                def _(j):
                    sum_v.at[r, pl.ds(j, 16)][...] = (
                        bufs[0].at[r, pl.ds(j, 16)][...]
                        + bufs[1].at[r, pl.ds(EMB + j, 16)][...])

            pltpu.sync_copy(
                sum_v,
                out_hbm.at[pl.ds(w * EX_PER_W + q * EX_PER_Q, EX_PER_Q)])

    return k(x3, seg1, table2)


def _tc_head(sums, x, W1, b1, W2, b2, Wo_p, bo_p):
    blk = 512
    grid = (B // blk,)

    def body(sums_ref, x_ref, w1_ref, b1_ref, w2_ref, b2_ref, wo_ref, bo_ref,
             out_ref):
        cnt = jnp.sum((x_ref[...] != 0).astype(jnp.float32), axis=1,
                      keepdims=True)
        pooled = sums_ref[...] / jnp.maximum(cnt, 1.0)
        h = jnp.dot(pooled, w1_ref[...], preferred_element_type=jnp.float32,
                    precision=lax.Precision.HIGHEST) + b1_ref[...]
        h = jnp.maximum(h, 0.0)
        h = jnp.dot(h, w2_ref[...], preferred_element_type=jnp.float32,
                    precision=lax.Precision.HIGHEST) + b2_ref[...]
        h = jnp.maximum(h, 0.0)
        out_ref[...] = jnp.dot(h, wo_ref[...],
                               preferred_element_type=jnp.float32,
                               precision=lax.Precision.HIGHEST) + bo_ref[...]

    return pl.pallas_call(
        body,
        grid=grid,
        in_specs=[
            pl.BlockSpec((blk, EMB), lambda i: (i, 0)),
            pl.BlockSpec((blk, L), lambda i: (i, 0)),
            pl.BlockSpec((EMB, HID), lambda i: (0, 0)),
            pl.BlockSpec((1, HID), lambda i: (0, 0)),
            pl.BlockSpec((HID, HID), lambda i: (0, 0)),
            pl.BlockSpec((1, HID), lambda i: (0, 0)),
            pl.BlockSpec((HID, 128), lambda i: (0, 0)),
            pl.BlockSpec((1, 128), lambda i: (0, 0)),
        ],
        out_specs=pl.BlockSpec((blk, 128), lambda i: (i, 0)),
        out_shape=jax.ShapeDtypeStruct((B, 128), jnp.float32),
    )(sums, x, W1, b1, W2, b2, Wo_p, bo_p)


def kernel(x, emb_table, W1, b1, W2, b2, Wout, bout):
    x = x.astype(jnp.int32)
    x3 = x.reshape(NW, NCHUNK, CHUNK)
    seg1 = (jnp.arange(EX_PER_Q * L, dtype=jnp.int32) // L).reshape(
        NCHUNK_Q, CHUNK)

    tail_rows = emb_table[NUNIT * CHUNK:].reshape(UNIT_TAIL // 2, 128)
    table2 = _sc_format(emb_table.T, tail_rows)
    sums = _sc_pooled_sums(x3, seg1, table2)

    Wo_p = jnp.zeros((HID, 128), jnp.float32).at[:, :TAGS].set(Wout)
    bo_p = jnp.zeros((1, 128), jnp.float32).at[:, :TAGS].set(bout[None, :])
    scores_p = _tc_head(sums, x, W1, b1[None, :], W2, b2[None, :], Wo_p, bo_p)
    return scores_p[:, :TAGS]


# speedup vs baseline: 1.5804x; 1.5804x over previous
import functools

import jax
import jax.numpy as jnp
from jax import lax
from jax.experimental import pallas as pl
from jax.experimental.pallas import tpu as pltpu
from jax.experimental.pallas import tpu_sc as plsc

B, L, EMB, HID, TAGS = 4096, 200, 64, 256, 5
V = 1000000
NC, NS = 2, 16
NW = NC * NS
EX_PER_W = B // NW
TOK_PER_W = EX_PER_W * L
CHUNK = 128
NCHUNK = TOK_PER_W // CHUNK
NQ = 4
EX_PER_Q = EX_PER_W // NQ
NCHUNK_Q = NCHUNK // NQ
NBUF = 5
ACC_ROWS = NS * EX_PER_Q

NUNIT = V // CHUNK
UNIT_TAIL = V - NUNIT * CHUNK
NITER = NUNIT // NW


def _sc_format(tabT, tail_rows):
    mesh = plsc.VectorSubcoreMesh(core_axis_name="c", subcore_axis_name="s")

    @functools.partial(
        pl.kernel,
        out_type=jax.ShapeDtypeStruct((V // 2, 128), jnp.float32),
        mesh=mesh,
        scratch_types=[
            pltpu.VMEM((EMB, CHUNK), jnp.float32),
            pltpu.VMEM((EMB, CHUNK), jnp.float32),
            pltpu.VMEM((EMB, CHUNK), jnp.float32),
            pltpu.VMEM((EMB, CHUNK), jnp.float32),
            *([pltpu.SemaphoreType.DMA] * 4),
        ],
        compiler_params=pltpu.CompilerParams(needs_layout_passes=False),
    )
    def k(tab_hbm, tail_hbm, out_hbm, inb0, inb1, outb0, outb1,
          is0, is1, os0, os1):
        w = lax.axis_index("s") * NC + lax.axis_index("c")

        dim_idx = [lax.iota(jnp.int32, 16) + 16 * kk for kk in range(4)]

        def transpose(inb, outb, nids, off=0):
            @plsc.parallel_loop(0, nids // 2, unroll=8)
            def _(orow):
                c0 = (off + 2 * orow).astype(jnp.int32)
                for kk in range(4):
                    vL = plsc.load_gather(
                        inb, [dim_idx[kk], jnp.broadcast_to(c0, (16,))])
                    outb.at[orow, pl.ds(16 * kk, 16)][...] = vL
                    vR = plsc.load_gather(
                        inb, [dim_idx[kk], jnp.broadcast_to(c0 + 1, (16,))])
                    outb.at[orow, pl.ds(EMB + 16 * kk, 16)][...] = vR

        def unit_of(i):
            return w + NW * i

        def in_copy(i, inb, sem):
            off = pl.multiple_of(unit_of(i) * CHUNK, CHUNK)
            return pltpu.async_copy(
                tab_hbm.at[:, pl.ds(off, CHUNK)], inb, sem)

        def out_copy(i, outb, sem):
            off = pl.multiple_of(unit_of(i) * (CHUNK // 2), CHUNK // 2)
            return pltpu.async_copy(
                outb, out_hbm.at[pl.ds(off, CHUNK // 2)], sem)

        in_copy(0, inb0, is0)
        in_copy(1, inb1, is1)
        out_copy(0, outb0, os0)
        out_copy(1, outb1, os1)

        @pl.loop(0, NITER, step=2)
        def _(i0):
            for (i_off, inb, outb, isem, osem) in (
                    (0, inb0, outb0, is0, os0), (1, inb1, outb1, is1, os1)):
                i = i0 + i_off
                pltpu.make_async_copy(
                    tab_hbm.at[:, pl.ds(0, CHUNK)], inb, isem).wait()
                pltpu.make_async_copy(
                    outb, out_hbm.at[pl.ds(0, CHUNK // 2)], osem).wait()
                transpose(inb, outb, CHUNK)

                @pl.when(i + 2 < NITER)
                def _():
                    in_copy(i + 2, inb, isem)

                out_copy(i, outb, osem)

        pltpu.make_async_copy(
            outb0, out_hbm.at[pl.ds(0, CHUNK // 2)], os0).wait()
        pltpu.make_async_copy(
            outb1, out_hbm.at[pl.ds(0, CHUNK // 2)], os1).wait()

        u_last = w + NW * NITER

        @pl.when(u_last < NUNIT)
        def _():
            off_in = pl.multiple_of(u_last * CHUNK, CHUNK)
            off_out = pl.multiple_of(u_last * (CHUNK // 2), CHUNK // 2)
            pltpu.sync_copy(tab_hbm.at[:, pl.ds(off_in, CHUNK)], inb0)
            transpose(inb0, outb0, CHUNK)
            pltpu.sync_copy(outb0, out_hbm.at[pl.ds(off_out, CHUNK // 2)])

        @pl.when(u_last == NUNIT)
        def _():
            pltpu.sync_copy(
                tail_hbm,
                out_hbm.at[pl.ds(NUNIT * (CHUNK // 2), UNIT_TAIL // 2)])

    return k(tabT, tail_rows)


def _sc_pooled_sums(x3, seg1, table2):
    mesh = plsc.VectorSubcoreMesh(core_axis_name="c", subcore_axis_name="s")

    @functools.partial(
        pl.kernel,
        out_type=jax.ShapeDtypeStruct((B, EMB), jnp.float32),
        mesh=mesh,
        scratch_types=[
            pltpu.VMEM((NCHUNK, CHUNK), jnp.int32),
            pltpu.VMEM((NCHUNK_Q, CHUNK), jnp.int32),
            pltpu.VMEM((NBUF, CHUNK), jnp.int32),
            pltpu.VMEM((NBUF, CHUNK), jnp.int32),
            pltpu.VMEM((EX_PER_Q, EMB), jnp.float32),
            *([pltpu.VMEM((CHUNK, 128), jnp.float32)] * NBUF),
            pltpu.VMEM_SHARED((2 * ACC_ROWS, 128), jnp.float32),
            *([pltpu.SemaphoreType.DMA] * (2 * NBUF)),
        ],
    )
    def k(x_hbm, seg_hbm, table_hbm, out_hbm, x_v, seg_v, gidx_v, sidx_v,
          sum_v, *rest):
        bufs = rest[:NBUF]
        acc_sh = rest[NBUF]
        gsem = rest[NBUF + 1:NBUF + 1 + NBUF]
        ssem = rest[NBUF + 1 + NBUF:]

        sid = lax.axis_index("s")
        w = sid * NC + lax.axis_index("c")
        pltpu.sync_copy(x_hbm.at[w], x_v)
        pltpu.sync_copy(seg_hbm, seg_v)

        zeros = jnp.zeros((16,), jnp.float32)
        seg_base = (sid * EX_PER_Q).astype(jnp.int32)

        def stage_indices(cx, cs, b):
            @pl.loop(0, CHUNK, step=16)
            def _(j):
                t = x_v.at[cx, pl.ds(j, 16)][...]
                gidx_v.at[b, pl.ds(j, 16)][...] = jax.lax.shift_right_logical(
                    t, 1)
                par = (t & 1) * ACC_ROWS
                sidx_v.at[b, pl.ds(j, 16)][...] = (
                    seg_v.at[cs, pl.ds(j, 16)][...] + par + seg_base)

        def zero_acc():
            @pl.loop(0, EX_PER_Q)
            def _(r):
                @pl.loop(0, 128, step=16)
                def _(j):
                    bufs[0].at[r, pl.ds(j, 16)][...] = zeros
            sl = bufs[0].at[pl.ds(0, EX_PER_Q)]
            pltpu.sync_copy(sl, acc_sh.at[pl.ds(sid * EX_PER_Q, EX_PER_Q)])
            pltpu.sync_copy(
                sl, acc_sh.at[pl.ds(ACC_ROWS + sid * EX_PER_Q, EX_PER_Q)])

        for q in range(NQ):
            zero_acc()
            base = q * NCHUNK_Q
            for b in range(NBUF):
                stage_indices(base + b, b, b)
                pltpu.async_copy(table_hbm.at[gidx_v.at[b]], bufs[b], gsem[b])

            @pl.loop(0, NCHUNK_Q, step=NBUF)
            def _(c0):
                scatters = []
                for b in range(NBUF):
                    pltpu.make_async_copy(
                        table_hbm.at[gidx_v.at[b]], bufs[b], gsem[b]).wait()
                    scatters.append(pltpu.async_copy(
                        bufs[b], acc_sh.at[sidx_v.at[b]], ssem[b], add=True))
                for b in range(NBUF):
                    scatters[b].wait()

                    @pl.when(c0 + NBUF + b < NCHUNK_Q)
                    def _():
                        stage_indices(base + c0 + NBUF + b, c0 + NBUF + b, b)
                        pltpu.async_copy(
                            table_hbm.at[gidx_v.at[b]], bufs[b], gsem[b])

            pltpu.sync_copy(acc_sh.at[pl.ds(sid * EX_PER_Q, EX_PER_Q)],
                            bufs[0].at[pl.ds(0, EX_PER_Q)])
            pltpu.sync_copy(
                acc_sh.at[pl.ds(ACC_ROWS + sid * EX_PER_Q, EX_PER_Q)],
                bufs[1].at[pl.ds(0, EX_PER_Q)])

            @pl.loop(0, EX_PER_Q)
            def _(r):
                @pl.loop(0, EMB, step=16)
                def _(j):
                    sum_v.at[r, pl.ds(j, 16)][...] = (
                        bufs[0].at[r, pl.ds(j, 16)][...]
                        + bufs[1].at[r, pl.ds(EMB + j, 16)][...])

            pltpu.sync_copy(
                sum_v,
                out_hbm.at[pl.ds(w * EX_PER_W + q * EX_PER_Q, EX_PER_Q)])

    return k(x3, seg1, table2)


def _tc_head(sums, x, W1, b1, W2, b2, Wo_p, bo_p):
    blk = 512
    grid = (B // blk,)

    def body(sums_ref, x_ref, w1_ref, b1_ref, w2_ref, b2_ref, wo_ref, bo_ref,
             out_ref):
        cnt = jnp.sum((x_ref[...] != 0).astype(jnp.float32), axis=1,
                      keepdims=True)
        pooled = sums_ref[...] / jnp.maximum(cnt, 1.0)
        h = jnp.dot(pooled, w1_ref[...], preferred_element_type=jnp.float32,
                    precision=lax.Precision.HIGHEST) + b1_ref[...]
        h = jnp.maximum(h, 0.0)
        h = jnp.dot(h, w2_ref[...], preferred_element_type=jnp.float32,
                    precision=lax.Precision.HIGHEST) + b2_ref[...]
        h = jnp.maximum(h, 0.0)
        out_ref[...] = jnp.dot(h, wo_ref[...],
                               preferred_element_type=jnp.float32,
                               precision=lax.Precision.HIGHEST) + bo_ref[...]

    return pl.pallas_call(
        body,
        grid=grid,
        in_specs=[
            pl.BlockSpec((blk, EMB), lambda i: (i, 0)),
            pl.BlockSpec((blk, L), lambda i: (i, 0)),
            pl.BlockSpec((EMB, HID), lambda i: (0, 0)),
            pl.BlockSpec((1, HID), lambda i: (0, 0)),
            pl.BlockSpec((HID, HID), lambda i: (0, 0)),
            pl.BlockSpec((1, HID), lambda i: (0, 0)),
            pl.BlockSpec((HID, 128), lambda i: (0, 0)),
            pl.BlockSpec((1, 128), lambda i: (0, 0)),
        ],
        out_specs=pl.BlockSpec((blk, 128), lambda i: (i, 0)),
        out_shape=jax.ShapeDtypeStruct((B, 128), jnp.float32),
    )(sums, x, W1, b1, W2, b2, Wo_p, bo_p)


def kernel(x, emb_table, W1, b1, W2, b2, Wout, bout):
    x = x.astype(jnp.int32)
    x3 = x.reshape(NW, NCHUNK, CHUNK)
    seg1 = (jnp.arange(EX_PER_Q * L, dtype=jnp.int32) // L).reshape(
        NCHUNK_Q, CHUNK)

    tail_rows = emb_table[NUNIT * CHUNK:].reshape(UNIT_TAIL // 2, 128)
    table2 = _sc_format(emb_table.T, tail_rows)
    sums = _sc_pooled_sums(x3, seg1, table2)

    Wo_p = jnp.zeros((HID, 128), jnp.float32).at[:, :TAGS].set(Wout)
    bo_p = jnp.zeros((1, 128), jnp.float32).at[:, :TAGS].set(bout[None, :])
    scores_p = _tc_head(sums, x, W1, b1[None, :], W2, b2[None, :], Wo_p, bo_p)
    return scores_p[:, :TAGS]


# speedup vs baseline: 2.3129x; 1.4634x over previous
import functools

import jax
import jax.numpy as jnp
from jax import lax
from jax.experimental import pallas as pl
from jax.experimental.pallas import tpu as pltpu
from jax.experimental.pallas import tpu_sc as plsc

B, L, EMB, HID, TAGS = 4096, 200, 64, 256, 5
NC, NS = 2, 16
NW = NC * NS
EX_PER_W = B // NW
TOK_PER_W = EX_PER_W * L
CHUNK = 128
NCHUNK = TOK_PER_W // CHUNK
NBUF = 5


def _sc_pooled_sums(x3, seg3, emb_table):
    mesh = plsc.VectorSubcoreMesh(core_axis_name="c", subcore_axis_name="s")

    @functools.partial(
        pl.kernel,
        out_type=jax.ShapeDtypeStruct((B, EMB), jnp.float32),
        mesh=mesh,
        scratch_types=[
            pltpu.VMEM((NCHUNK, CHUNK), jnp.int32),
            pltpu.VMEM((NCHUNK, CHUNK), jnp.int32),
            *([pltpu.VMEM((CHUNK, EMB), jnp.float32)] * NBUF),
            pltpu.VMEM_SHARED((NS * EX_PER_W, EMB), jnp.float32),
            *([pltpu.SemaphoreType.DMA] * (2 * NBUF)),
        ],
        compiler_params=pltpu.CompilerParams(use_tc_tiling_on_sc=False),
    )
    def k(x_hbm, seg_hbm, table_hbm, out_hbm, x_v, seg_v, *rest):
        bufs = rest[:NBUF]
        acc_sh = rest[NBUF]
        gsem = rest[NBUF + 1:NBUF + 1 + NBUF]
        ssem = rest[NBUF + 1 + NBUF:]

        sid = lax.axis_index("s")
        w = sid * NC + lax.axis_index("c")
        pltpu.sync_copy(x_hbm.at[w], x_v)
        pltpu.sync_copy(seg_hbm.at[sid], seg_v)

        zeros = jnp.zeros((16,), jnp.float32)

        @pl.loop(0, CHUNK)
        def _(r):
            @pl.loop(0, EMB, step=16)
            def _(j):
                bufs[0].at[r, pl.ds(j, 16)][...] = zeros

        pltpu.sync_copy(bufs[0], acc_sh.at[pl.ds(sid * EX_PER_W, EX_PER_W)])

        for b in range(NBUF):
            pltpu.async_copy(table_hbm.at[x_v.at[b]], bufs[b], gsem[b])

        @pl.loop(0, NCHUNK, step=NBUF)
        def _(c0):
            scatters = []
            for b in range(NBUF):
                pltpu.make_async_copy(
                    table_hbm.at[x_v.at[0]], bufs[b], gsem[b]).wait()
                scatters.append(pltpu.async_copy(
                    bufs[b], acc_sh.at[seg_v.at[c0 + b]], ssem[b], add=True))
            for b in range(NBUF):
                scatters[b].wait()

                @pl.when(c0 + NBUF + b < NCHUNK)
                def _():
                    pltpu.async_copy(
                        table_hbm.at[x_v.at[c0 + NBUF + b]], bufs[b], gsem[b])

        pltpu.sync_copy(acc_sh.at[pl.ds(sid * EX_PER_W, EX_PER_W)],
                        out_hbm.at[pl.ds(w * EX_PER_W, EX_PER_W)])

    return k(x3, seg3, emb_table)


def _tc_head(sums, x, W1, b1, W2, b2, Wo_p, bo_p):
    blk = 512
    grid = (B // blk,)

    def body(sums_ref, x_ref, w1_ref, b1_ref, w2_ref, b2_ref, wo_ref, bo_ref,
             out_ref):
        cnt = jnp.sum((x_ref[...] != 0).astype(jnp.float32), axis=1,
                      keepdims=True)
        pooled = sums_ref[...] / jnp.maximum(cnt, 1.0)
        h = jnp.dot(pooled, w1_ref[...], preferred_element_type=jnp.float32,
                    precision=lax.Precision.HIGHEST) + b1_ref[...]
        h = jnp.maximum(h, 0.0)
        h = jnp.dot(h, w2_ref[...], preferred_element_type=jnp.float32,
                    precision=lax.Precision.HIGHEST) + b2_ref[...]
        h = jnp.maximum(h, 0.0)
        out_ref[...] = jnp.dot(h, wo_ref[...],
                               preferred_element_type=jnp.float32,
                               precision=lax.Precision.HIGHEST) + bo_ref[...]

    return pl.pallas_call(
        body,
        grid=grid,
        in_specs=[
            pl.BlockSpec((blk, EMB), lambda i: (i, 0)),
            pl.BlockSpec((blk, L), lambda i: (i, 0)),
            pl.BlockSpec((EMB, HID), lambda i: (0, 0)),
            pl.BlockSpec((1, HID), lambda i: (0, 0)),
            pl.BlockSpec((HID, HID), lambda i: (0, 0)),
            pl.BlockSpec((1, HID), lambda i: (0, 0)),
            pl.BlockSpec((HID, 128), lambda i: (0, 0)),
            pl.BlockSpec((1, 128), lambda i: (0, 0)),
        ],
        out_specs=pl.BlockSpec((blk, 128), lambda i: (i, 0)),
        out_shape=jax.ShapeDtypeStruct((B, 128), jnp.float32),
    )(sums, x, W1, b1, W2, b2, Wo_p, bo_p)


def kernel(x, emb_table, W1, b1, W2, b2, Wout, bout):
    x = x.astype(jnp.int32)
    x3 = x.reshape(NW, NCHUNK, CHUNK)
    seg2 = (jnp.arange(TOK_PER_W, dtype=jnp.int32) // L).reshape(NCHUNK, CHUNK)
    seg3 = seg2[None, :, :] + (
        jnp.arange(NS, dtype=jnp.int32) * EX_PER_W)[:, None, None]

    sums = _sc_pooled_sums(x3, seg3, emb_table)

    Wo_p = jnp.zeros((HID, 128), jnp.float32).at[:, :TAGS].set(Wout)
    bo_p = jnp.zeros((1, 128), jnp.float32).at[:, :TAGS].set(bout[None, :])
    scores_p = _tc_head(sums, x, W1, b1[None, :], W2, b2[None, :], Wo_p, bo_p)
    return scores_p[:, :TAGS]


# speedup vs baseline: 2.3533x; 1.0175x over previous
import functools

import jax
import jax.numpy as jnp
from jax import lax
from jax.experimental import pallas as pl
from jax.experimental.pallas import tpu as pltpu
from jax.experimental.pallas import tpu_sc as plsc

B, L, EMB, HID, TAGS = 4096, 200, 64, 256, 5
NC, NS = 2, 16
NW = NC * NS
EX_PER_W = B // NW
TOK_PER_W = EX_PER_W * L
CHUNK = 128
NCHUNK = TOK_PER_W // CHUNK
NBUF = 8


def _sc_pooled_sums(x3, seg3, emb_table):
    mesh = plsc.VectorSubcoreMesh(core_axis_name="c", subcore_axis_name="s")

    @functools.partial(
        pl.kernel,
        out_type=jax.ShapeDtypeStruct((B, EMB), jnp.float32),
        mesh=mesh,
        scratch_types=[
            pltpu.VMEM((NCHUNK, CHUNK), jnp.int32),
            pltpu.VMEM((NCHUNK, CHUNK), jnp.int32),
            *([pltpu.VMEM((CHUNK, EMB), jnp.float32)] * NBUF),
            pltpu.VMEM_SHARED((NS * EX_PER_W, EMB), jnp.float32),
            *([pltpu.SemaphoreType.DMA] * (2 * NBUF)),
        ],
        compiler_params=pltpu.CompilerParams(use_tc_tiling_on_sc=False),
    )
    def k(x_hbm, seg_hbm, table_hbm, out_hbm, x_v, seg_v, *rest):
        bufs = rest[:NBUF]
        acc_sh = rest[NBUF]
        gsem = rest[NBUF + 1:NBUF + 1 + NBUF]
        ssem = rest[NBUF + 1 + NBUF:]

        sid = lax.axis_index("s")
        w = sid * NC + lax.axis_index("c")
        pltpu.sync_copy(x_hbm.at[w], x_v)
        pltpu.sync_copy(seg_hbm.at[sid], seg_v)

        zeros = jnp.zeros((16,), jnp.float32)

        @pl.loop(0, CHUNK)
        def _(r):
            @pl.loop(0, EMB, step=16)
            def _(j):
                bufs[0].at[r, pl.ds(j, 16)][...] = zeros

        pltpu.sync_copy(bufs[0], acc_sh.at[pl.ds(sid * EX_PER_W, EX_PER_W)])

        for b in range(NBUF):
            pltpu.async_copy(table_hbm.at[x_v.at[b]], bufs[b], gsem[b])

        @pl.loop(0, NCHUNK, step=NBUF)
        def _(c0):
            scatters = []
            for b in range(NBUF):
                pltpu.make_async_copy(
                    table_hbm.at[x_v.at[0]], bufs[b], gsem[b]).wait()
                scatters.append(pltpu.async_copy(
                    bufs[b], acc_sh.at[seg_v.at[c0 + b]], ssem[b], add=True))
            for b in range(NBUF):
                scatters[b].wait()

                @pl.when(c0 + NBUF + b < NCHUNK)
                def _():
                    pltpu.async_copy(
                        table_hbm.at[x_v.at[c0 + NBUF + b]], bufs[b], gsem[b])

        pltpu.sync_copy(acc_sh.at[pl.ds(sid * EX_PER_W, EX_PER_W)],
                        out_hbm.at[pl.ds(w * EX_PER_W, EX_PER_W)])

    return k(x3, seg3, emb_table)


def _tc_head(sums, x, W1, b1, W2, b2, Wo_p, bo_p):
    blk = 512
    grid = (B // blk,)

    def body(sums_ref, x_ref, w1_ref, b1_ref, w2_ref, b2_ref, wo_ref, bo_ref,
             out_ref):
        cnt = jnp.sum((x_ref[...] != 0).astype(jnp.float32), axis=1,
                      keepdims=True)
        pooled = sums_ref[...] / jnp.maximum(cnt, 1.0)
        h = jnp.dot(pooled, w1_ref[...], preferred_element_type=jnp.float32,
                    precision=lax.Precision.HIGHEST) + b1_ref[...]
        h = jnp.maximum(h, 0.0)
        h = jnp.dot(h, w2_ref[...], preferred_element_type=jnp.float32,
                    precision=lax.Precision.HIGHEST) + b2_ref[...]
        h = jnp.maximum(h, 0.0)
        out_ref[...] = jnp.dot(h, wo_ref[...],
                               preferred_element_type=jnp.float32,
                               precision=lax.Precision.HIGHEST) + bo_ref[...]

    return pl.pallas_call(
        body,
        grid=grid,
        in_specs=[
            pl.BlockSpec((blk, EMB), lambda i: (i, 0)),
            pl.BlockSpec((blk, L), lambda i: (i, 0)),
            pl.BlockSpec((EMB, HID), lambda i: (0, 0)),
            pl.BlockSpec((1, HID), lambda i: (0, 0)),
            pl.BlockSpec((HID, HID), lambda i: (0, 0)),
            pl.BlockSpec((1, HID), lambda i: (0, 0)),
            pl.BlockSpec((HID, 128), lambda i: (0, 0)),
            pl.BlockSpec((1, 128), lambda i: (0, 0)),
        ],
        out_specs=pl.BlockSpec((blk, 128), lambda i: (i, 0)),
        out_shape=jax.ShapeDtypeStruct((B, 128), jnp.float32),
    )(sums, x, W1, b1, W2, b2, Wo_p, bo_p)


def kernel(x, emb_table, W1, b1, W2, b2, Wout, bout):
    x = x.astype(jnp.int32)
    x3 = x.reshape(NW, NCHUNK, CHUNK)
    seg2 = (jnp.arange(TOK_PER_W, dtype=jnp.int32) // L).reshape(NCHUNK, CHUNK)
    seg3 = seg2[None, :, :] + (
        jnp.arange(NS, dtype=jnp.int32) * EX_PER_W)[:, None, None]

    sums = _sc_pooled_sums(x3, seg3, emb_table)

    Wo_p = jnp.zeros((HID, 128), jnp.float32).at[:, :TAGS].set(Wout)
    bo_p = jnp.zeros((1, 128), jnp.float32).at[:, :TAGS].set(bout[None, :])
    scores_p = _tc_head(sums, x, W1, b1[None, :], W2, b2[None, :], Wo_p, bo_p)
    return scores_p[:, :TAGS]
